# Initial kernel scaffold; baseline (speedup 1.0000x reference)
import jax
import jax.numpy as jnp
from jax.experimental import pallas as pl


def kernel(mask, instance_ids, quaternion, scales, xy, z):
    raise NotImplementedError("write your pallas kernel here")


# speedup vs baseline: 4.7893x; 4.7893x over previous
import jax
import jax.numpy as jnp
from jax.experimental import pallas as pl
from jax.experimental.pallas import tpu as pltpu

_B, _H, _W, _KP = 4, 256, 256, 16
_RH = 64


def _agg_body(ids_ref, mask_ref, q_ref, s_ref, xy_ref, z_ref,
              imask_ref, xyout_ref, stats_ref, acc_ref, accm_ref):
    b = pl.program_id(0)
    r = pl.program_id(1)
    nr = pl.num_programs(1)

    @pl.when(r == 0)
    def _init():
        acc_ref[...] = jnp.zeros_like(acc_ref)
        accm_ref[...] = jnp.zeros_like(accm_ref)

    ids = ids_ref[0]
    mcls = mask_ref[0]
    xy0 = xy_ref[0, 0]
    xy1 = xy_ref[0, 1]
    chans = ([q_ref[0, c] for c in range(4)]
             + [s_ref[0, c] for c in range(3)]
             + [z_ref[0]])
    base = b * _KP + 1
    for j in range(_KP):
        bm = ids == (base + j)
        bmf = bm.astype(jnp.float32)
        imask_ref[j] = bmf
        xyout_ref[j, 0] = bmf * xy0
        xyout_ref[j, 1] = bmf * xy1
        for c in range(8):
            acc_ref[j, c] += jnp.sum(bmf * chans[c], axis=0)
        acc_ref[j, 8] += jnp.sum(bmf, axis=0)
        cm = jnp.max(jnp.where(bm, mcls, 0), axis=0)
        accm_ref[j] = jnp.maximum(accm_ref[j], cm)

    @pl.when(r == nr - 1)
    def _fin():
        red = jnp.sum(acc_ref[...], axis=-1)
        cnt = red[:, 8:9]
        qm = red[:, 0:4] / cnt
        sm = red[:, 4:7] / cnt
        zm = red[:, 7:8] / cnt
        qn = qm / jnp.sqrt(jnp.sum(qm * qm, axis=1, keepdims=True))
        ze = jnp.exp(zm)
        cls = jnp.max(accm_ref[...], axis=-1).astype(jnp.float32)[:, None]
        out9 = jnp.concatenate([qn, sm, ze, cls], axis=1)
        stats_ref[...] = jnp.concatenate(
            [out9, jnp.zeros((_KP, 128 - 9), jnp.float32)], axis=1)


def _run(mask, instance_ids, quaternion, scales, xy, z, interpret=False):
    grid = (_B, _H // _RH)
    out_shapes = (
        jax.ShapeDtypeStruct((_B * _KP, _H, _W), jnp.float32),
        jax.ShapeDtypeStruct((_B * _KP, 2, _H, _W), jnp.float32),
        jax.ShapeDtypeStruct((_B * _KP, 128), jnp.float32),
    )
    return pl.pallas_call(
        _agg_body,
        grid=grid,
        in_specs=[
            pl.BlockSpec((1, _RH, _W), lambda b, r: (b, r, 0)),
            pl.BlockSpec((1, _RH, _W), lambda b, r: (b, r, 0)),
            pl.BlockSpec((1, 4, _RH, _W), lambda b, r: (b, 0, r, 0)),
            pl.BlockSpec((1, 3, _RH, _W), lambda b, r: (b, 0, r, 0)),
            pl.BlockSpec((1, 2, _RH, _W), lambda b, r: (b, 0, r, 0)),
            pl.BlockSpec((1, _RH, _W), lambda b, r: (b, r, 0)),
        ],
        out_specs=[
            pl.BlockSpec((_KP, _RH, _W), lambda b, r: (b, r, 0)),
            pl.BlockSpec((_KP, 2, _RH, _W), lambda b, r: (b, 0, r, 0)),
            pl.BlockSpec((_KP, 128), lambda b, r: (b, 0)),
        ],
        out_shape=out_shapes,
        scratch_shapes=[
            pltpu.VMEM((_KP, 9, _W), jnp.float32),
            pltpu.VMEM((_KP, _W), jnp.int32),
        ],
        interpret=interpret,
    )(instance_ids, mask, quaternion, scales, xy, z)


@jax.jit
def kernel(mask, instance_ids, quaternion, scales, xy, z):
    imask, xyout, stats = _run(mask, instance_ids, quaternion, scales, xy, z)
    cls = stats[:, 8].astype(jnp.int32)
    qn = stats[:, 0:4]
    sm = stats[:, 4:7]
    ze = stats[:, 7:8]
    sample_ids = jnp.repeat(jnp.arange(_B, dtype=jnp.int32), _KP)
    return (cls, imask, sample_ids, qn, sm, xyout, ze)
